# baseline (speedup 1.0000x reference)
import functools

import jax
import jax.numpy as jnp
from jax import lax
from jax.experimental import pallas as pl
from jax.experimental.pallas import tpu as pltpu
from jax.experimental.pallas import tpu_sc as plsc

_K = 8192
_D = 256
_N = 16384
_BETA = 0.25
_NH = _N // 2

_BN1 = 512
_BN3 = 1024

_NC = 2
_NS = 16
_NW = _NC * _NS
_BPW = _NH // _NW
_CH = 256

_NCH = 16
_KC = _K // _NCH


def _argmin_body(lat_ref, w_ref, ssl_ref, ssw_ref, colf_ref, _oh_in_ref,
                 inds_ref, oh_ref):
    l2 = -2.0 * lat_ref[...]
    ssl = ssl_ref[...]
    dmin = None
    iminf = None
    for c in range(_NCH):
        w_c = w_ref[c * _KC:(c + 1) * _KC, :]
        mm2 = jax.lax.dot_general(
            l2, w_c, (((1,), (1,)), ((), ())),
            preferred_element_type=jnp.float32)
        d = (ssl + ssw_ref[:, c * _KC:(c + 1) * _KC]) + mm2
        dmin_c = jnp.min(d, axis=1, keepdims=True)
        colf_c = colf_ref[:, c * _KC:(c + 1) * _KC]
        imin_c = jnp.min(
            jnp.where(d == dmin_c, colf_c, jnp.float32(_K)),
            axis=1, keepdims=True)
        if c == 0:
            dmin, iminf = dmin_c, imin_c
        else:
            iminf = jnp.where(dmin_c < dmin, imin_c, iminf)
            dmin = jnp.minimum(dmin, dmin_c)
    inds_ref[0, 0, :] = iminf[:, 0].astype(jnp.int32)
    for c in range(_NCH):
        oh_ref[:, c * _KC:(c + 1) * _KC] = (
            colf_ref[:, c * _KC:(c + 1) * _KC] == iminf).astype(jnp.float32)


def _argmin_half(latents, W, ssl, ssw, colf, oh_buf, half, alias):
    nbh = _NH // _BN1
    off = half * nbh
    inds3, oh = pl.pallas_call(
        _argmin_body,
        grid=(nbh,),
        in_specs=[
            pl.BlockSpec((_BN1, _D), lambda i: (i + off, 0)),
            pl.BlockSpec((_K, _D), lambda i: (0, 0)),
            pl.BlockSpec((_BN1, 1), lambda i: (i + off, 0)),
            pl.BlockSpec((1, _K), lambda i: (0, 0)),
            pl.BlockSpec((1, _K), lambda i: (0, 0)),
            pl.BlockSpec((8, 128), lambda i: (0, 0)),
        ],
        out_specs=[
            pl.BlockSpec((1, 1, _BN1), lambda i: (i, 0, 0)),
            pl.BlockSpec((_BN1, _K), lambda i: (i + off, 0)),
        ],
        out_shape=[
            jax.ShapeDtypeStruct((nbh, 1, _BN1), jnp.int32),
            jax.ShapeDtypeStruct((_N, _K), jnp.float32),
        ],
        input_output_aliases={5: 1} if alias else {},
    )(latents, W, ssl, ssw, colf, oh_buf)
    return inds3.reshape(_NH), oh


@functools.partial(
    pl.kernel,
    mesh=plsc.VectorSubcoreMesh(core_axis_name="c", subcore_axis_name="s"),
    out_type=jax.ShapeDtypeStruct((_NH, _D), jnp.float32),
    scratch_types=[
        pltpu.VMEM((_BPW,), jnp.int32),
        pltpu.VMEM((_CH, _D), jnp.float32),
        pltpu.SemaphoreType.DMA,
    ],
)
def _sc_gather(w_hbm, idx_hbm, q_hbm, idx_v, rows_v, sem):
    wid = lax.axis_index("s") * _NC + lax.axis_index("c")
    base = wid * _BPW
    pltpu.sync_copy(idx_hbm.at[pl.ds(base, _BPW)], idx_v)
    for c in range(_BPW // _CH):
        pltpu.async_copy(
            w_hbm.at[idx_v.at[pl.ds(c * _CH, _CH)]], rows_v, sem).wait()
        pltpu.sync_copy(rows_v, q_hbm.at[pl.ds(base + c * _CH, _CH)])


def _quant_body(lat_ref, q_ref, _quant_in_ref, quant_ref, loss_ref):
    l = lat_ref[...]
    q = q_ref[...]
    quant_ref[...] = l + (q - l)
    loss_ref[...] = jnp.sum((q - l) ** 2).reshape(1, 1, 1)


def _quant_half(latents, q, quant_buf, half, alias):
    nbh = _NH // _BN3
    off = half * nbh
    quant, loss_parts = pl.pallas_call(
        _quant_body,
        grid=(nbh,),
        in_specs=[
            pl.BlockSpec((_BN3, _D), lambda i: (i + off, 0)),
            pl.BlockSpec((_BN3, _D), lambda i: (i, 0)),
            pl.BlockSpec((8, 128), lambda i: (0, 0)),
        ],
        out_specs=[
            pl.BlockSpec((_BN3, _D), lambda i: (i + off, 0)),
            pl.BlockSpec((1, 1, 1), lambda i: (i, 0, 0)),
        ],
        out_shape=[
            jax.ShapeDtypeStruct((_N, _D), jnp.float32),
            jax.ShapeDtypeStruct((nbh, 1, 1), jnp.float32),
        ],
        input_output_aliases={2: 0} if alias else {},
    )(latents, q, quant_buf)
    return quant, loss_parts


def kernel(latents, W):
    n, d = latents.shape
    k = W.shape[0]
    ssl = jnp.sum(latents ** 2, axis=1, keepdims=True)
    ssw = jnp.sum(W ** 2, axis=1)[None, :]
    colf = jnp.arange(k, dtype=jnp.float32)[None, :]
    dummy = jnp.zeros((8, 128), jnp.float32)

    inds0, oh0 = _argmin_half(latents, W, ssl, ssw, colf, dummy, 0, False)
    q0 = _sc_gather(W, inds0)
    inds1, one_hot = _argmin_half(latents, W, ssl, ssw, colf, oh0, 1, True)
    quant0, parts0 = _quant_half(latents, q0, dummy, 0, False)
    q1 = _sc_gather(W, inds1)
    quant, parts1 = _quant_half(latents, q1, quant0, 1, True)

    m = (jnp.sum(parts0) + jnp.sum(parts1)) / (n * d)
    vq_loss = m * _BETA + m
    return (quant, vq_loss, one_hot)


# speedup vs baseline: 1.0611x; 1.0611x over previous
import functools

import jax
import jax.numpy as jnp
from jax import lax
from jax.experimental import pallas as pl
from jax.experimental.pallas import tpu as pltpu
from jax.experimental.pallas import tpu_sc as plsc

_K = 8192
_D = 256
_N = 16384
_BETA = 0.25
_NH = _N // 2

_BN1 = 512
_BN3 = 1024

_NC = 2
_NS = 16
_NW = _NC * _NS
_BPW = _NH // _NW
_CH = 256

_NCH = 16
_KC = _K // _NCH


def _argmin_body(lat_ref, w_ref, ssl_ref, ssw_ref, colf_ref, _oh_in_ref,
                 inds_ref, oh_ref):
    l2 = -2.0 * lat_ref[...]
    ssl = ssl_ref[...]
    bv = None
    bi = None
    for c in range(_NCH):
        w_c = w_ref[c * _KC:(c + 1) * _KC, :]
        mm2 = jax.lax.dot_general(
            l2, w_c, (((1,), (1,)), ((), ())),
            preferred_element_type=jnp.float32)
        d = (ssl + ssw_ref[:, c * _KC:(c + 1) * _KC]) + mm2
        if c == 0:
            bv = d
            bi = jnp.zeros_like(d)
        else:
            upd = d < bv
            bi = jnp.where(upd, jnp.float32(c), bi)
            bv = jnp.minimum(bv, d)
    gmin = jnp.min(bv, axis=1, keepdims=True)
    lane = colf_ref[:, 0:_KC]
    colv = bi * jnp.float32(_KC) + lane
    iminf = jnp.min(jnp.where(bv == gmin, colv, jnp.float32(_K)),
                    axis=1, keepdims=True)
    inds_ref[0, 0, :] = iminf[:, 0].astype(jnp.int32)
    for c in range(_NCH):
        oh_ref[:, c * _KC:(c + 1) * _KC] = (
            colf_ref[:, c * _KC:(c + 1) * _KC] == iminf).astype(jnp.float32)


def _argmin_half(latents, W, ssl, ssw, colf, oh_buf, half, alias):
    nbh = _NH // _BN1
    off = half * nbh
    inds3, oh = pl.pallas_call(
        _argmin_body,
        grid=(nbh,),
        in_specs=[
            pl.BlockSpec((_BN1, _D), lambda i: (i + off, 0)),
            pl.BlockSpec((_K, _D), lambda i: (0, 0)),
            pl.BlockSpec((_BN1, 1), lambda i: (i + off, 0)),
            pl.BlockSpec((1, _K), lambda i: (0, 0)),
            pl.BlockSpec((1, _K), lambda i: (0, 0)),
            pl.BlockSpec((8, 128), lambda i: (0, 0)),
        ],
        out_specs=[
            pl.BlockSpec((1, 1, _BN1), lambda i: (i, 0, 0)),
            pl.BlockSpec((_BN1, _K), lambda i: (i + off, 0)),
        ],
        out_shape=[
            jax.ShapeDtypeStruct((nbh, 1, _BN1), jnp.int32),
            jax.ShapeDtypeStruct((_N, _K), jnp.float32),
        ],
        input_output_aliases={5: 1} if alias else {},
    )(latents, W, ssl, ssw, colf, oh_buf)
    return inds3.reshape(_NH), oh


@functools.partial(
    pl.kernel,
    mesh=plsc.VectorSubcoreMesh(core_axis_name="c", subcore_axis_name="s"),
    out_type=jax.ShapeDtypeStruct((_NH, _D), jnp.float32),
    scratch_types=[
        pltpu.VMEM((_BPW,), jnp.int32),
        pltpu.VMEM((_CH, _D), jnp.float32),
        pltpu.SemaphoreType.DMA,
    ],
)
def _sc_gather(w_hbm, idx_hbm, q_hbm, idx_v, rows_v, sem):
    wid = lax.axis_index("s") * _NC + lax.axis_index("c")
    base = wid * _BPW
    pltpu.sync_copy(idx_hbm.at[pl.ds(base, _BPW)], idx_v)
    for c in range(_BPW // _CH):
        pltpu.async_copy(
            w_hbm.at[idx_v.at[pl.ds(c * _CH, _CH)]], rows_v, sem).wait()
        pltpu.sync_copy(rows_v, q_hbm.at[pl.ds(base + c * _CH, _CH)])


def _quant_body(lat_ref, q_ref, _quant_in_ref, quant_ref, loss_ref):
    l = lat_ref[...]
    q = q_ref[...]
    quant_ref[...] = l + (q - l)
    loss_ref[...] = jnp.sum((q - l) ** 2).reshape(1, 1, 1)


def _quant_half(latents, q, quant_buf, half, alias):
    nbh = _NH // _BN3
    off = half * nbh
    quant, loss_parts = pl.pallas_call(
        _quant_body,
        grid=(nbh,),
        in_specs=[
            pl.BlockSpec((_BN3, _D), lambda i: (i + off, 0)),
            pl.BlockSpec((_BN3, _D), lambda i: (i, 0)),
            pl.BlockSpec((8, 128), lambda i: (0, 0)),
        ],
        out_specs=[
            pl.BlockSpec((_BN3, _D), lambda i: (i + off, 0)),
            pl.BlockSpec((1, 1, 1), lambda i: (i, 0, 0)),
        ],
        out_shape=[
            jax.ShapeDtypeStruct((_N, _D), jnp.float32),
            jax.ShapeDtypeStruct((nbh, 1, 1), jnp.float32),
        ],
        input_output_aliases={2: 0} if alias else {},
    )(latents, q, quant_buf)
    return quant, loss_parts


def kernel(latents, W):
    n, d = latents.shape
    k = W.shape[0]
    ssl = jnp.sum(latents ** 2, axis=1, keepdims=True)
    ssw = jnp.sum(W ** 2, axis=1)[None, :]
    colf = jnp.arange(k, dtype=jnp.float32)[None, :]
    dummy = jnp.zeros((8, 128), jnp.float32)

    inds0, oh0 = _argmin_half(latents, W, ssl, ssw, colf, dummy, 0, False)
    q0 = _sc_gather(W, inds0)
    inds1, one_hot = _argmin_half(latents, W, ssl, ssw, colf, oh0, 1, True)
    quant0, parts0 = _quant_half(latents, q0, dummy, 0, False)
    q1 = _sc_gather(W, inds1)
    quant, parts1 = _quant_half(latents, q1, quant0, 1, True)

    m = (jnp.sum(parts0) + jnp.sum(parts1)) / (n * d)
    vq_loss = m * _BETA + m
    return (quant, vq_loss, one_hot)


# speedup vs baseline: 1.0919x; 1.0290x over previous
import functools

import jax
import jax.numpy as jnp
from jax import lax
from jax.experimental import pallas as pl
from jax.experimental.pallas import tpu as pltpu
from jax.experimental.pallas import tpu_sc as plsc

_K = 8192
_D = 256
_N = 16384
_BETA = 0.25
_NH = _N // 2

_BN1 = 512
_BN3 = 1024

_NC = 2
_NS = 16
_NW = _NC * _NS
_BPW = _NH // _NW
_CH = 256

_NCH = 32
_KC = _K // _NCH


def _argmin_body(lat_ref, w_ref, ssl_ref, ssw_ref, colf_ref, _oh_in_ref,
                 inds_ref, oh_ref):
    l2 = -2.0 * lat_ref[...]
    ssl = ssl_ref[...]
    bv = None
    bi = None
    for c in range(_NCH):
        w_c = w_ref[c * _KC:(c + 1) * _KC, :]
        mm2 = jax.lax.dot_general(
            l2, w_c, (((1,), (1,)), ((), ())),
            preferred_element_type=jnp.float32)
        d = (ssl + ssw_ref[:, c * _KC:(c + 1) * _KC]) + mm2
        if c == 0:
            bv = d
            bi = jnp.zeros_like(d)
        else:
            upd = d < bv
            bi = jnp.where(upd, jnp.float32(c), bi)
            bv = jnp.minimum(bv, d)
    gmin = jnp.min(bv, axis=1, keepdims=True)
    lane = colf_ref[:, 0:_KC]
    colv = bi * jnp.float32(_KC) + lane
    iminf = jnp.min(jnp.where(bv == gmin, colv, jnp.float32(_K)),
                    axis=1, keepdims=True)
    inds_ref[0, 0, :] = iminf[:, 0].astype(jnp.int32)
    for c in range(_NCH):
        oh_ref[:, c * _KC:(c + 1) * _KC] = (
            colf_ref[:, c * _KC:(c + 1) * _KC] == iminf).astype(jnp.float32)


def _argmin_half(latents, W, ssl, ssw, colf, oh_buf, half, alias):
    nbh = _NH // _BN1
    off = half * nbh
    inds3, oh = pl.pallas_call(
        _argmin_body,
        grid=(nbh,),
        in_specs=[
            pl.BlockSpec((_BN1, _D), lambda i: (i + off, 0)),
            pl.BlockSpec((_K, _D), lambda i: (0, 0)),
            pl.BlockSpec((_BN1, 1), lambda i: (i + off, 0)),
            pl.BlockSpec((1, _K), lambda i: (0, 0)),
            pl.BlockSpec((1, _K), lambda i: (0, 0)),
            pl.BlockSpec((8, 128), lambda i: (0, 0)),
        ],
        out_specs=[
            pl.BlockSpec((1, 1, _BN1), lambda i: (i, 0, 0)),
            pl.BlockSpec((_BN1, _K), lambda i: (i + off, 0)),
        ],
        out_shape=[
            jax.ShapeDtypeStruct((nbh, 1, _BN1), jnp.int32),
            jax.ShapeDtypeStruct((_N, _K), jnp.float32),
        ],
        input_output_aliases={5: 1} if alias else {},
    )(latents, W, ssl, ssw, colf, oh_buf)
    return inds3.reshape(_NH), oh


@functools.partial(
    pl.kernel,
    mesh=plsc.VectorSubcoreMesh(core_axis_name="c", subcore_axis_name="s"),
    out_type=jax.ShapeDtypeStruct((_NH, _D), jnp.float32),
    scratch_types=[
        pltpu.VMEM((_BPW,), jnp.int32),
        pltpu.VMEM((_CH, _D), jnp.float32),
        pltpu.SemaphoreType.DMA,
    ],
)
def _sc_gather(w_hbm, idx_hbm, q_hbm, idx_v, rows_v, sem):
    wid = lax.axis_index("s") * _NC + lax.axis_index("c")
    base = wid * _BPW
    pltpu.sync_copy(idx_hbm.at[pl.ds(base, _BPW)], idx_v)
    for c in range(_BPW // _CH):
        pltpu.async_copy(
            w_hbm.at[idx_v.at[pl.ds(c * _CH, _CH)]], rows_v, sem).wait()
        pltpu.sync_copy(rows_v, q_hbm.at[pl.ds(base + c * _CH, _CH)])


def _quant_body(lat_ref, q_ref, _quant_in_ref, quant_ref, loss_ref):
    l = lat_ref[...]
    q = q_ref[...]
    quant_ref[...] = l + (q - l)
    loss_ref[...] = jnp.sum((q - l) ** 2).reshape(1, 1, 1)


def _quant_half(latents, q, quant_buf, half, alias):
    nbh = _NH // _BN3
    off = half * nbh
    quant, loss_parts = pl.pallas_call(
        _quant_body,
        grid=(nbh,),
        in_specs=[
            pl.BlockSpec((_BN3, _D), lambda i: (i + off, 0)),
            pl.BlockSpec((_BN3, _D), lambda i: (i, 0)),
            pl.BlockSpec((8, 128), lambda i: (0, 0)),
        ],
        out_specs=[
            pl.BlockSpec((_BN3, _D), lambda i: (i + off, 0)),
            pl.BlockSpec((1, 1, 1), lambda i: (i, 0, 0)),
        ],
        out_shape=[
            jax.ShapeDtypeStruct((_N, _D), jnp.float32),
            jax.ShapeDtypeStruct((nbh, 1, 1), jnp.float32),
        ],
        input_output_aliases={2: 0} if alias else {},
    )(latents, q, quant_buf)
    return quant, loss_parts


def kernel(latents, W):
    n, d = latents.shape
    k = W.shape[0]
    ssl = jnp.sum(latents ** 2, axis=1, keepdims=True)
    ssw = jnp.sum(W ** 2, axis=1)[None, :]
    colf = jnp.arange(k, dtype=jnp.float32)[None, :]
    dummy = jnp.zeros((8, 128), jnp.float32)

    inds0, oh0 = _argmin_half(latents, W, ssl, ssw, colf, dummy, 0, False)
    q0 = _sc_gather(W, inds0)
    inds1, one_hot = _argmin_half(latents, W, ssl, ssw, colf, oh0, 1, True)
    quant0, parts0 = _quant_half(latents, q0, dummy, 0, False)
    q1 = _sc_gather(W, inds1)
    quant, parts1 = _quant_half(latents, q1, quant0, 1, True)

    m = (jnp.sum(parts0) + jnp.sum(parts1)) / (n * d)
    vq_loss = m * _BETA + m
    return (quant, vq_loss, one_hot)
